# Initial kernel scaffold; baseline (speedup 1.0000x reference)
import functools

import jax
import jax.numpy as jnp
from jax import lax
from jax.experimental import pallas as pl
from jax.experimental.pallas import tpu as pltpu
from jax.experimental.pallas import tpu_sc as plsc

MARGIN = 0.5
NUM_ANCHORS = 2048
D = 256
NC = 2
NS = 16
NW = NC * NS
L = 16
DSL = D // L


def _vsqrt(x):
    i = lax.bitcast_convert_type(x, jnp.int32)
    i = jnp.int32(0x5F3759DF) - lax.shift_right_arithmetic(i, 1)
    y = lax.bitcast_convert_type(i, jnp.float32)
    for _ in range(3):
        y = y * (1.5 - 0.5 * x * y * y)
    return x * y


def _make_sc_kernel(total_tokens):
    tpw = total_tokens // NW
    chunk = 64
    nch = tpw // chunk
    mesh = plsc.VectorSubcoreMesh(core_axis_name="c", subcore_axis_name="s")

    @functools.partial(
        pl.kernel,
        out_type=jax.ShapeDtypeStruct((NW, 2 * L), jnp.float32),
        mesh=mesh,
        scratch_types=[
            pltpu.VMEM((tpw,), jnp.int32),
            pltpu.VMEM((tpw,), jnp.int32),
            pltpu.VMEM((chunk, D), jnp.float32),
            pltpu.VMEM((chunk, D), jnp.float32),
            pltpu.VMEM((chunk,), jnp.float32),
            pltpu.VMEM((2 * L,), jnp.float32),
            pltpu.SemaphoreType.DMA,
            pltpu.SemaphoreType.DMA,
        ],
    )
    def sc_kernel(z_hbm, ids_hbm, cent_hbm, out_hbm,
                  idx_raw, idx_cl, z_v, cent_v, d2_v, acc_v, sem_z, sem_c):
        wid = lax.axis_index("c") * NS + lax.axis_index("s")
        base = wid * tpw

        pltpu.sync_copy(ids_hbm.at[pl.ds(base, tpw)], idx_raw)

        def clamp_body(j, _):
            v = idx_raw[pl.ds(j * L, L)]
            idx_cl[pl.ds(j * L, L)] = jnp.maximum(v, 0)
            return _
        lax.fori_loop(0, tpw // L, clamp_body, None)

        zero = jnp.zeros((L,), jnp.float32)
        acc_v[pl.ds(0, L)] = zero
        acc_v[pl.ds(L, L)] = zero

        def chunk_body(g, _):
            tok0 = g * chunk
            cp_z = pltpu.make_async_copy(
                z_hbm.at[pl.ds(base + tok0, chunk)], z_v, sem_z)
            cp_c = pltpu.make_async_copy(
                cent_hbm.at[idx_cl.at[pl.ds(tok0, chunk)]], cent_v, sem_c)
            cp_z.start()
            cp_c.start()
            cp_z.wait()
            cp_c.wait()

            def tok_body(t, _):
                d = z_v[t, pl.ds(0, L)] - cent_v[t, pl.ds(0, L)]
                acc = d * d
                for k in range(1, DSL):
                    d = z_v[t, pl.ds(k * L, L)] - cent_v[t, pl.ds(k * L, L)]
                    acc = acc + d * d
                d2_v[t] = jnp.sum(acc)
                return _
            lax.fori_loop(0, chunk, tok_body, None)

            for j in range(chunk // L):
                x = jnp.maximum(d2_v[pl.ds(j * L, L)], 0.0)
                raw = idx_raw[pl.ds(tok0 + j * L, L)]
                valid = raw >= 0
                h = jnp.maximum(_vsqrt(x) - MARGIN, 0.0)
                plsc.addupdate(acc_v.at[pl.ds(0, L)],
                               jnp.where(valid, h, 0.0))
                plsc.addupdate(acc_v.at[pl.ds(L, L)],
                               jnp.where(valid, 1.0, 0.0))
            return _
        lax.fori_loop(0, nch, chunk_body, None)

        pltpu.sync_copy(acc_v, out_hbm.at[wid])

    return sc_kernel


@jax.jit
def kernel(z_q, root_ids, centroids):
    b, s, d = z_q.shape
    total = b * s
    z2 = z_q.reshape(total, d)
    ids = root_ids.reshape(total)
    parts = _make_sc_kernel(total)(z2, ids, centroids)
    hinge_total = jnp.sum(parts[:, :16])
    count = jnp.sum(parts[:, 16:])
    return jnp.where(count > 0, hinge_total / jnp.maximum(count, 1.0), 0.0)


# speedup vs baseline: 1.7272x; 1.7272x over previous
import functools

import jax
import jax.numpy as jnp
from jax import lax
from jax.experimental import pallas as pl
from jax.experimental.pallas import tpu as pltpu
from jax.experimental.pallas import tpu_sc as plsc

MARGIN = 0.5
NUM_ANCHORS = 2048
D = 256
NC = 2
NS = 16
NW = NC * NS
L = 16
DSL = D // L


def _vsqrt(x):
    i = lax.bitcast_convert_type(x, jnp.int32)
    i = jnp.int32(0x5F3759DF) - lax.shift_right_arithmetic(i, 1)
    y = lax.bitcast_convert_type(i, jnp.float32)
    for _ in range(3):
        y = y * (1.5 - 0.5 * x * y * y)
    return x * y


def _make_sc_kernel(total_tokens):
    tpw = total_tokens // NW
    chunk = 64
    nch = tpw // chunk
    mesh = plsc.VectorSubcoreMesh(
        core_axis_name="c", subcore_axis_name="s",
        num_cores=NC, num_subcores=NS)

    @functools.partial(
        pl.kernel,
        out_type=jax.ShapeDtypeStruct((NW, 2 * L), jnp.float32),
        mesh=mesh,
        compiler_params=pltpu.CompilerParams(needs_layout_passes=False),
        scratch_types=[
            pltpu.VMEM((tpw,), jnp.int32),
            pltpu.VMEM((tpw,), jnp.int32),
            pltpu.VMEM((chunk, D), jnp.float32),
            pltpu.VMEM((chunk, D), jnp.float32),
            pltpu.VMEM((L * L,), jnp.float32),
            pltpu.VMEM((2 * L,), jnp.float32),
            pltpu.SemaphoreType.DMA,
            pltpu.SemaphoreType.DMA,
        ],
    )
    def sc_kernel(z_hbm, ids_hbm, cent_hbm, out_hbm,
                  idx_raw, idx_cl, z_v, cent_v, m_v, acc_v, sem_z, sem_c):
        wid = lax.axis_index("c") * NS + lax.axis_index("s")
        base = wid * tpw

        pltpu.sync_copy(ids_hbm.at[pl.ds(base, tpw)], idx_raw)

        def clamp_body(j, _):
            v = idx_raw[pl.ds(j * L, L)]
            idx_cl[pl.ds(j * L, L)] = jnp.maximum(v, 0)
            return _
        lax.fori_loop(0, tpw // L, clamp_body, None)

        zero = jnp.zeros((L,), jnp.float32)
        acc_v[pl.ds(0, L)] = zero
        acc_v[pl.ds(L, L)] = zero

        def chunk_body(g, _):
            tok0 = g * chunk
            cp_z = pltpu.make_async_copy(
                z_hbm.at[pl.ds(base + tok0, chunk)], z_v, sem_z)
            cp_c = pltpu.make_async_copy(
                cent_hbm.at[idx_cl.at[pl.ds(tok0, chunk)]], cent_v, sem_c)
            cp_z.start()
            cp_c.start()
            cp_z.wait()
            cp_c.wait()

            rows = lax.iota(jnp.int32, L)

            def grp_body(q, _):
                t0 = q * L
                for i in range(L):
                    t = t0 + i
                    d = z_v[t, pl.ds(0, L)] - cent_v[t, pl.ds(0, L)]
                    acc = d * d
                    for k in range(1, DSL):
                        d = (z_v[t, pl.ds(k * L, L)]
                             - cent_v[t, pl.ds(k * L, L)])
                        acc = acc + d * d
                    m_v[pl.ds(i * L, L)] = acc
                x = plsc.load_gather(m_v, [rows * L])
                for j in range(1, L):
                    x = x + plsc.load_gather(m_v, [rows * L + j])
                raw = idx_raw[pl.ds(tok0 + t0, L)]
                valid = raw >= 0
                h = jnp.maximum(_vsqrt(x) - MARGIN, 0.0)
                plsc.addupdate(acc_v.at[pl.ds(0, L)],
                               jnp.where(valid, h, 0.0))
                plsc.addupdate(acc_v.at[pl.ds(L, L)],
                               jnp.where(valid, 1.0, 0.0))
                return _
            lax.fori_loop(0, chunk // L, grp_body, None)
            return _
        lax.fori_loop(0, nch, chunk_body, None)

        pltpu.sync_copy(acc_v, out_hbm.at[wid])

    return sc_kernel


@jax.jit
def kernel(z_q, root_ids, centroids):
    b, s, d = z_q.shape
    total = b * s
    z2 = z_q.reshape(total, d)
    ids = root_ids.reshape(total)
    parts = _make_sc_kernel(total)(z2, ids, centroids)
    hinge_total = jnp.sum(parts[:, :16])
    count = jnp.sum(parts[:, 16:])
    return jnp.where(count > 0, hinge_total / jnp.maximum(count, 1.0), 0.0)


# speedup vs baseline: 2.3101x; 1.3375x over previous
import functools

import jax
import jax.numpy as jnp
from jax import lax
from jax.experimental import pallas as pl
from jax.experimental.pallas import tpu as pltpu
from jax.experimental.pallas import tpu_sc as plsc

MARGIN = 0.5
NUM_ANCHORS = 2048
D = 256
NC = 2
NS = 16
NW = NC * NS
L = 16
DSL = D // L


def _vsqrt(x):
    i = lax.bitcast_convert_type(x, jnp.int32)
    i = jnp.int32(0x5F3759DF) - lax.shift_right_arithmetic(i, 1)
    y = lax.bitcast_convert_type(i, jnp.float32)
    for _ in range(3):
        y = y * (1.5 - 0.5 * x * y * y)
    return x * y


def _make_sc_kernel(total_tokens):
    tpw = total_tokens // NW
    chunk = 64
    nch = tpw // chunk
    mesh = plsc.VectorSubcoreMesh(
        core_axis_name="c", subcore_axis_name="s",
        num_cores=NC, num_subcores=NS)

    @functools.partial(
        pl.kernel,
        out_type=jax.ShapeDtypeStruct((NW, 2 * L), jnp.float32),
        mesh=mesh,
        compiler_params=pltpu.CompilerParams(needs_layout_passes=False),
        scratch_types=[
            pltpu.VMEM((tpw,), jnp.int32),
            pltpu.VMEM((tpw,), jnp.int32),
            pltpu.VMEM((2, chunk, D), jnp.float32),
            pltpu.VMEM((2, chunk, D), jnp.float32),
            pltpu.VMEM((L * L,), jnp.float32),
            pltpu.VMEM((2 * L,), jnp.float32),
            pltpu.SemaphoreType.DMA,
            pltpu.SemaphoreType.DMA,
            pltpu.SemaphoreType.DMA,
            pltpu.SemaphoreType.DMA,
        ],
    )
    def sc_kernel(z_hbm, ids_hbm, cent_hbm, out_hbm,
                  idx_raw, idx_cl, z_v, cent_v, m_v, acc_v,
                  sem_z0, sem_z1, sem_c0, sem_c1):
        wid = lax.axis_index("c") * NS + lax.axis_index("s")
        base = wid * tpw

        pltpu.sync_copy(ids_hbm.at[pl.ds(base, tpw)], idx_raw)

        def clamp_body(j, _):
            v = idx_raw[pl.ds(j * L, L)]
            idx_cl[pl.ds(j * L, L)] = jnp.maximum(v, 0)
            return _
        lax.fori_loop(0, tpw // L, clamp_body, None)

        zero = jnp.zeros((L,), jnp.float32)
        acc_v[pl.ds(0, L)] = zero
        acc_v[pl.ds(L, L)] = zero

        sems = ((sem_z0, sem_c0), (sem_z1, sem_c1))
        rows = lax.iota(jnp.int32, L)

        def copies(g, buf):
            sz, sc = sems[buf]
            return (
                pltpu.make_async_copy(
                    z_hbm.at[pl.ds(base + g * chunk, chunk)],
                    z_v.at[buf], sz),
                pltpu.make_async_copy(
                    cent_hbm.at[idx_cl.at[pl.ds(g * chunk, chunk)]],
                    cent_v.at[buf], sc),
            )

        def issue(g, buf):
            for cp in copies(g, buf):
                cp.start()

        def process(g, buf, last):
            tok0 = g * chunk
            for cp in copies(g, buf):
                cp.wait()
            zb = z_v.at[buf]
            cb = cent_v.at[buf]

            def grp_body(q, _):
                t0 = q * L
                for i in range(L):
                    t = t0 + i
                    d = zb[t, pl.ds(0, L)] - cb[t, pl.ds(0, L)]
                    acc = d * d
                    for k in range(1, DSL):
                        d = (zb[t, pl.ds(k * L, L)]
                             - cb[t, pl.ds(k * L, L)])
                        acc = acc + d * d
                    m_v[pl.ds(i * L, L)] = acc
                x = plsc.load_gather(m_v, [rows * L])
                for j in range(1, L):
                    x = x + plsc.load_gather(m_v, [rows * L + j])
                raw = idx_raw[pl.ds(tok0 + t0, L)]
                valid = raw >= 0
                h = jnp.maximum(_vsqrt(x) - MARGIN, 0.0)
                plsc.addupdate(acc_v.at[pl.ds(0, L)],
                               jnp.where(valid, h, 0.0))
                plsc.addupdate(acc_v.at[pl.ds(L, L)],
                               jnp.where(valid, 1.0, 0.0))
                return _
            lax.fori_loop(0, chunk // L, grp_body, None)

            @pl.when(jnp.logical_not(last))
            def _issue_next():
                issue(g + 2, buf)

        issue(0, 0)
        issue(1, 1)

        def pair_body(p, _):
            last = p >= nch // 2 - 1
            process(2 * p, 0, last)
            process(2 * p + 1, 1, last)
            return _
        lax.fori_loop(0, nch // 2, pair_body, None)

        pltpu.sync_copy(acc_v, out_hbm.at[wid])

    return sc_kernel


@jax.jit
def kernel(z_q, root_ids, centroids):
    b, s, d = z_q.shape
    total = b * s
    z2 = z_q.reshape(total, d)
    ids = root_ids.reshape(total)
    parts = _make_sc_kernel(total)(z2, ids, centroids)
    hinge_total = jnp.sum(parts[:, :16])
    count = jnp.sum(parts[:, 16:])
    return jnp.where(count > 0, hinge_total / jnp.maximum(count, 1.0), 0.0)


# speedup vs baseline: 2.6153x; 1.1321x over previous
import functools

import jax
import jax.numpy as jnp
from jax import lax
from jax.experimental import pallas as pl
from jax.experimental.pallas import tpu as pltpu
from jax.experimental.pallas import tpu_sc as plsc

MARGIN = 0.5
NUM_ANCHORS = 2048
D = 256
NC = 2
NS = 16
NW = NC * NS
L = 16
DSL = D // L
DSL2 = D // (2 * L)


def _vsqrt(x):
    i = lax.bitcast_convert_type(x, jnp.int32)
    i = jnp.int32(0x5F3759DF) - lax.shift_right_arithmetic(i, 1)
    y = lax.bitcast_convert_type(i, jnp.float32)
    for _ in range(3):
        y = y * (1.5 - 0.5 * x * y * y)
    return x * y


def _make_sc_kernel(total_tokens):
    tpw = total_tokens // NW
    chunk = 64
    nch = tpw // chunk
    mesh = plsc.VectorSubcoreMesh(
        core_axis_name="c", subcore_axis_name="s",
        num_cores=NC, num_subcores=NS)

    @functools.partial(
        pl.kernel,
        out_type=jax.ShapeDtypeStruct((NW, 2 * L), jnp.float32),
        mesh=mesh,
        compiler_params=pltpu.CompilerParams(needs_layout_passes=False),
        scratch_types=[
            pltpu.VMEM((tpw,), jnp.int32),
            pltpu.VMEM((tpw,), jnp.int32),
            pltpu.VMEM((2, chunk, D), jnp.float32),
            pltpu.VMEM((2, chunk, D // 2), jnp.int32),
            pltpu.VMEM((L * L,), jnp.float32),
            pltpu.VMEM((2 * L,), jnp.float32),
            pltpu.SemaphoreType.DMA,
            pltpu.SemaphoreType.DMA,
            pltpu.SemaphoreType.DMA,
            pltpu.SemaphoreType.DMA,
        ],
    )
    def sc_kernel(z_hbm, ids_hbm, cent_hbm, out_hbm,
                  idx_raw, idx_cl, z_v, cent_v, m_v, acc_v,
                  sem_z0, sem_z1, sem_c0, sem_c1):
        wid = lax.axis_index("c") * NS + lax.axis_index("s")
        base = wid * tpw

        pltpu.sync_copy(ids_hbm.at[pl.ds(base, tpw)], idx_raw)

        def clamp_body(j, _):
            v = idx_raw[pl.ds(j * L, L)]
            idx_cl[pl.ds(j * L, L)] = jnp.maximum(v, 0)
            return _
        lax.fori_loop(0, tpw // L, clamp_body, None)

        zero = jnp.zeros((L,), jnp.float32)
        acc_v[pl.ds(0, L)] = zero
        acc_v[pl.ds(L, L)] = zero

        sems = ((sem_z0, sem_c0), (sem_z1, sem_c1))
        rows = lax.iota(jnp.int32, L)

        def copies(g, buf):
            sz, sc = sems[buf]
            return (
                pltpu.make_async_copy(
                    z_hbm.at[pl.ds(base + g * chunk, chunk)],
                    z_v.at[buf], sz),
                pltpu.make_async_copy(
                    cent_hbm.at[idx_cl.at[pl.ds(g * chunk, chunk)]],
                    cent_v.at[buf], sc),
            )

        def issue(g, buf):
            for cp in copies(g, buf):
                cp.start()

        def process(g, buf, last):
            tok0 = g * chunk
            for cp in copies(g, buf):
                cp.wait()
            zb = z_v.at[buf]
            cb = cent_v.at[buf]

            def grp_body(q, _):
                t0 = q * L
                for i in range(L):
                    t = t0 + i
                    acc = None
                    for k in range(DSL2):
                        zlo = zb[t, pl.ds(k * 2 * L, L)]
                        zhi = zb[t, pl.ds(k * 2 * L + L, L)]
                        cw = plsc.bitcast(cb[t, pl.ds(k * L, L)],
                                          jnp.bfloat16)
                        clo, chi = plsc.unpack(
                            cw, format=plsc.PackFormat.INTERLEAVED)
                        dlo = zlo - clo.astype(jnp.float32)
                        dhi = zhi - chi.astype(jnp.float32)
                        sq = dlo * dlo + dhi * dhi
                        acc = sq if acc is None else acc + sq
                    m_v[pl.ds(i * L, L)] = acc
                x = plsc.load_gather(m_v, [rows * L])
                for j in range(1, L):
                    x = x + plsc.load_gather(m_v, [rows * L + j])
                raw = idx_raw[pl.ds(tok0 + t0, L)]
                valid = raw >= 0
                h = jnp.maximum(_vsqrt(x) - MARGIN, 0.0)
                plsc.addupdate(acc_v.at[pl.ds(0, L)],
                               jnp.where(valid, h, 0.0))
                plsc.addupdate(acc_v.at[pl.ds(L, L)],
                               jnp.where(valid, 1.0, 0.0))
                return _
            lax.fori_loop(0, chunk // L, grp_body, None)

            @pl.when(jnp.logical_not(last))
            def _issue_next():
                issue(g + 2, buf)

        issue(0, 0)
        issue(1, 1)

        def pair_body(p, _):
            last = p >= nch // 2 - 1
            process(2 * p, 0, last)
            process(2 * p + 1, 1, last)
            return _
        lax.fori_loop(0, nch // 2, pair_body, None)

        pltpu.sync_copy(acc_v, out_hbm.at[wid])

    return sc_kernel


@jax.jit
def kernel(z_q, root_ids, centroids):
    b, s, d = z_q.shape
    total = b * s
    z2 = z_q.reshape(total, d)
    na = centroids.shape[0]
    cperm = (centroids.reshape(na, d // 32, 2, 16)
             .swapaxes(-1, -2).reshape(na, d))
    cpack = lax.bitcast_convert_type(
        cperm.astype(jnp.bfloat16).reshape(na, d // 2, 2), jnp.int32)
    ids = root_ids.reshape(total)
    parts = _make_sc_kernel(total)(z2, ids, cpack)
    hinge_total = jnp.sum(parts[:, :16])
    count = jnp.sum(parts[:, 16:])
    return jnp.where(count > 0, hinge_total / jnp.maximum(count, 1.0), 0.0)
